# Initial kernel scaffold; baseline (speedup 1.0000x reference)
import jax
import jax.numpy as jnp
from jax.experimental import pallas as pl


def kernel(ent, rel, h, r, t):
    raise NotImplementedError("write your pallas kernel here")


# speedup vs baseline: 2.0365x; 2.0365x over previous
import functools
import jax
import jax.numpy as jnp
from jax import lax
from jax.experimental import pallas as pl
from jax.experimental.pallas import tpu as pltpu, tpu_sc as plsc

DIM = 128
BATCH = 16384
CHUNK = 64
LANES = 16


def _score_body(ent_hbm, rel_hbm, h_idx, r_idx, t_idx, out_hbm,
                hix, rix, tix, hbuf, rbuf, tbuf, outv, hsem, rsem, tsem):
    nc = 2
    wid = lax.axis_index("s") * nc + lax.axis_index("c")
    per_w = BATCH // 32
    nchunks = per_w // CHUNK

    pltpu.sync_copy(h_idx.at[wid], hix)
    pltpu.sync_copy(r_idx.at[wid], rix)
    pltpu.sync_copy(t_idx.at[wid], tix)

    iota = lax.iota(jnp.int32, LANES)

    for c in range(nchunks):
        cp_h = pltpu.make_async_copy(ent_hbm.at[hix.at[c]], hbuf, hsem)
        cp_r = pltpu.make_async_copy(rel_hbm.at[rix.at[c]], rbuf, rsem)
        cp_t = pltpu.make_async_copy(ent_hbm.at[tix.at[c]], tbuf, tsem)
        cp_h.start(); cp_r.start(); cp_t.start()
        cp_h.wait(); cp_r.wait(); cp_t.wait()

        for g in range(CHUNK // LANES):
            def body(j, vec):
                row = g * LANES + j
                acc = jnp.zeros((LANES,), jnp.float32)
                for k in range(DIM // LANES):
                    sl_r = pl.ds(k * LANES, LANES)
                    sl_i = pl.ds(DIM + k * LANES, LANES)
                    hr = hbuf[row, sl_r]; hi = hbuf[row, sl_i]
                    rr = rbuf[row, sl_r]; ri = rbuf[row, sl_i]
                    tr = tbuf[row, sl_r]; ti = tbuf[row, sl_i]
                    acc = acc + hr * (rr * tr + ri * ti) + hi * (rr * ti - ri * tr)
                for s in (8, 4, 2, 1):
                    acc = acc + lax.gather(
                        acc, (iota ^ s)[:, None],
                        dimension_numbers=lax.GatherDimensionNumbers(
                            offset_dims=(), collapsed_slice_dims=(0,),
                            start_index_map=(0,)),
                        slice_sizes=(1,),
                        mode=lax.GatherScatterMode.PROMISE_IN_BOUNDS)
                return jnp.where(iota == j, acc, vec)

            vec = lax.fori_loop(0, LANES, body, jnp.zeros((LANES,), jnp.float32))
            outv[pl.ds(c * CHUNK + g * LANES, LANES)] = vec

    pltpu.sync_copy(outv, out_hbm.at[pl.ds(wid * per_w, per_w)])


def kernel(ent, rel, h, r, t):
    per_w = BATCH // 32
    h3 = h.astype(jnp.int32).reshape(32, per_w // CHUNK, CHUNK)
    r3 = r.astype(jnp.int32).reshape(32, per_w // CHUNK, CHUNK)
    t3 = t.astype(jnp.int32).reshape(32, per_w // CHUNK, CHUNK)

    mesh = plsc.VectorSubcoreMesh(core_axis_name="c", subcore_axis_name="s")
    run = functools.partial(
        pl.kernel,
        mesh=mesh,
        out_type=jax.ShapeDtypeStruct((BATCH,), jnp.float32),
        scratch_types=[
            pltpu.VMEM((per_w // CHUNK, CHUNK), jnp.int32),
            pltpu.VMEM((per_w // CHUNK, CHUNK), jnp.int32),
            pltpu.VMEM((per_w // CHUNK, CHUNK), jnp.int32),
            pltpu.VMEM((CHUNK, 2 * DIM), jnp.float32),
            pltpu.VMEM((CHUNK, 2 * DIM), jnp.float32),
            pltpu.VMEM((CHUNK, 2 * DIM), jnp.float32),
            pltpu.VMEM((per_w,), jnp.float32),
            pltpu.SemaphoreType.DMA,
            pltpu.SemaphoreType.DMA,
            pltpu.SemaphoreType.DMA,
        ],
    )(_score_body)
    return run(ent, rel, h3, r3, t3)


# speedup vs baseline: 2.5242x; 1.2395x over previous
import functools
import jax
import jax.numpy as jnp
from jax import lax
from jax.experimental import pallas as pl
from jax.experimental.pallas import tpu as pltpu, tpu_sc as plsc

DIM = 128
BATCH = 16384
CHUNK = 64
LANES = 16

_GDN = lax.GatherDimensionNumbers(
    offset_dims=(), collapsed_slice_dims=(0,), start_index_map=(0,))


def _shuffle(x, idx):
    return lax.gather(x, idx[:, None], dimension_numbers=_GDN,
                      slice_sizes=(1,),
                      mode=lax.GatherScatterMode.PROMISE_IN_BOUNDS)


def _score_body(ent_hbm, rel_hbm, h_idx, r_idx, t_idx, out_hbm,
                hix, rix, tix, hbuf, rbuf, tbuf, outv,
                hsem0, rsem0, tsem0, hsem1, rsem1, tsem1):
    sems = ((hsem0, rsem0, tsem0), (hsem1, rsem1, tsem1))
    nc = 2
    wid = lax.axis_index("s") * nc + lax.axis_index("c")
    per_w = BATCH // 32
    nchunks = per_w // CHUNK

    pltpu.sync_copy(h_idx.at[wid], hix)
    pltpu.sync_copy(r_idx.at[wid], rix)
    pltpu.sync_copy(t_idx.at[wid], tix)

    iota = lax.iota(jnp.int32, LANES)

    def start(c, b):
        hs, rs, ts = sems[b]
        cp_h = pltpu.make_async_copy(ent_hbm.at[hix.at[c]], hbuf.at[b], hs)
        cp_r = pltpu.make_async_copy(rel_hbm.at[rix.at[c]], rbuf.at[b], rs)
        cp_t = pltpu.make_async_copy(ent_hbm.at[tix.at[c]], tbuf.at[b], ts)
        cp_h.start(); cp_r.start(); cp_t.start()
        return cp_h, cp_r, cp_t

    pending = start(0, 0)
    for c in range(nchunks):
        b = c & 1
        nxt = start(c + 1, 1 - b) if c + 1 < nchunks else None
        for cp in pending:
            cp.wait()

        for g in range(CHUNK // LANES):
            def body(j, vec):
                acc = jnp.zeros((LANES,), jnp.float32)
                for k in range(DIM // LANES):
                    sl_r = pl.ds(k * LANES, LANES)
                    sl_i = pl.ds(DIM + k * LANES, LANES)
                    hr = hbuf[b, j, sl_r]; hi = hbuf[b, j, sl_i]
                    rr = rbuf[b, j, sl_r]; ri = rbuf[b, j, sl_i]
                    tr = tbuf[b, j, sl_r]; ti = tbuf[b, j, sl_i]
                    acc = acc + hr * (rr * tr + ri * ti) + hi * (rr * ti - ri * tr)
                for s in (8, 4, 2, 1):
                    acc = acc + _shuffle(acc, iota ^ s)
                return jnp.where(iota == (j - g * LANES), acc, vec)

            vec = lax.fori_loop(g * LANES, (g + 1) * LANES, body,
                                jnp.zeros((LANES,), jnp.float32))
            outv[pl.ds(c * CHUNK + g * LANES, LANES)] = vec
        pending = nxt

    pltpu.sync_copy(outv, out_hbm.at[pl.ds(wid * per_w, per_w)])


def kernel(ent, rel, h, r, t):
    per_w = BATCH // 32
    h3 = h.astype(jnp.int32).reshape(32, per_w // CHUNK, CHUNK)
    r3 = r.astype(jnp.int32).reshape(32, per_w // CHUNK, CHUNK)
    t3 = t.astype(jnp.int32).reshape(32, per_w // CHUNK, CHUNK)

    mesh = plsc.VectorSubcoreMesh(core_axis_name="c", subcore_axis_name="s")
    run = functools.partial(
        pl.kernel,
        mesh=mesh,
        out_type=jax.ShapeDtypeStruct((BATCH,), jnp.float32),
        scratch_types=[
            pltpu.VMEM((per_w // CHUNK, CHUNK), jnp.int32),
            pltpu.VMEM((per_w // CHUNK, CHUNK), jnp.int32),
            pltpu.VMEM((per_w // CHUNK, CHUNK), jnp.int32),
            pltpu.VMEM((2, CHUNK, 2 * DIM), jnp.float32),
            pltpu.VMEM((2, CHUNK, 2 * DIM), jnp.float32),
            pltpu.VMEM((2, CHUNK, 2 * DIM), jnp.float32),
            pltpu.VMEM((per_w,), jnp.float32),
            pltpu.SemaphoreType.DMA,
            pltpu.SemaphoreType.DMA,
            pltpu.SemaphoreType.DMA,
            pltpu.SemaphoreType.DMA,
            pltpu.SemaphoreType.DMA,
            pltpu.SemaphoreType.DMA,
        ],
    )(_score_body)
    return run(ent, rel, h3, r3, t3)
